# Initial kernel scaffold; baseline (speedup 1.0000x reference)
import jax
import jax.numpy as jnp
from jax.experimental import pallas as pl


def kernel(att_ws):
    raise NotImplementedError("write your pallas kernel here")


# speedup vs baseline: 1.0218x; 1.0218x over previous
import jax
import jax.numpy as jnp
from jax.experimental import pallas as pl
from jax.experimental.pallas import tpu as pltpu


def _body(x_ref, dur_ref, focus_ref, best_ref, am_ref):
    h = pl.program_id(0)
    x = x_ref[0]
    Tf, Tt = x.shape
    rowmax = jnp.max(x, axis=1, keepdims=True)
    score = jnp.sum(rowmax)
    ids = jax.lax.broadcasted_iota(jnp.int32, (Tf, Tt), 1)
    am = jnp.min(jnp.where(x == rowmax, ids, Tt), axis=1, keepdims=True)

    @pl.when((h == 0) | (score > best_ref[0]))
    def _():
        best_ref[0] = score
        am_ref[...] = am

    @pl.when(h == pl.num_programs(0) - 1)
    def _():
        a = am_ref[...]
        onehot = (a == ids).astype(jnp.int32)
        dur_ref[...] = jnp.sum(onehot, axis=0, keepdims=True)
        focus_ref[0] = best_ref[0] / Tf


def kernel(att_ws):
    L, H, Tf, Tt = att_ws.shape
    NH = L * H
    flat = att_ws.reshape(NH, Tf, Tt)
    dur, focus = pl.pallas_call(
        _body,
        grid=(NH,),
        in_specs=[pl.BlockSpec((1, Tf, Tt), lambda h: (h, 0, 0))],
        out_specs=[
            pl.BlockSpec((1, Tt), lambda h: (0, 0)),
            pl.BlockSpec(memory_space=pltpu.SMEM),
        ],
        out_shape=[
            jax.ShapeDtypeStruct((1, Tt), jnp.int32),
            jax.ShapeDtypeStruct((1,), jnp.float32),
        ],
        scratch_shapes=[
            pltpu.SMEM((1,), jnp.float32),
            pltpu.VMEM((Tf, 1), jnp.int32),
        ],
    )(flat)
    durations = dur[0].astype(jnp.int64)
    focus_rate = focus[0]
    return durations, focus_rate
